# baseline (speedup 1.0000x reference)
import functools

import jax
import jax.numpy as jnp
from jax import lax
from jax.experimental import pallas as pl
from jax.experimental.pallas import tpu as pltpu
from jax.experimental.pallas import tpu_sc as plsc

N = 2000
E_EDGES = 32000
H1, F1 = 8, 16
D_HID = H1 * F1
DB = 400
NEG = -1e30


def _layer1_pre_body(x_ref, w1_ref, h_ref):
    h_ref[...] = jnp.dot(x_ref[...], w1_ref[...],
                         preferred_element_type=jnp.float32)


def _head_proj(att_ref):
    att = att_ref[...]
    tiled = jnp.concatenate([att] * H1, axis=1)
    row = lax.broadcasted_iota(jnp.int32, (H1, D_HID), 0)
    col = lax.broadcasted_iota(jnp.int32, (H1, D_HID), 1)
    return jnp.where(col // F1 == row, tiled, 0.0)


def _attn1_body(c_ref, h_ref, hblk_ref, asrc_ref, adst_ref, b1_ref, out_ref):
    c = c_ref[...]
    h1 = h_ref[...]
    asr = lax.dot_general(_head_proj(asrc_ref), h1, (((1,), (1,)), ((), ())),
                          preferred_element_type=jnp.float32)
    adc = lax.dot_general(hblk_ref[...], _head_proj(adst_ref),
                          (((1,), (1,)), ((), ())),
                          preferred_element_type=jnp.float32)
    haug = jnp.concatenate(
        [h1, jnp.ones((N, 1), jnp.float32)], axis=1).astype(jnp.bfloat16)
    for hh in range(H1):
        a_s = asr[hh : hh + 1, :]
        a_d = adc[:, hh : hh + 1]
        ag = jnp.max(a_s, axis=1, keepdims=True)
        t = a_d + ag
        bnd = jnp.where(t >= 0.0, t, 0.2 * t)
        u = jnp.exp(a_d - bnd)
        up = jnp.exp(0.2 * a_d - bnd)
        v = jnp.exp(a_s)
        vp = jnp.exp(0.2 * a_s)
        p = (c * jnp.maximum(u * v, up * vp)).astype(jnp.bfloat16)
        o_aug = jnp.dot(
            p, jnp.concatenate(
                [haug[:, hh * F1 : (hh + 1) * F1], haug[:, D_HID:]], axis=1),
            preferred_element_type=jnp.float32)
        o = (o_aug[:, :F1] / (o_aug[:, F1 : F1 + 1] + 1e-16)
             + b1_ref[:, hh * F1 : (hh + 1) * F1])
        out_ref[:, hh * F1 : (hh + 1) * F1] = jnp.where(
            o > 0.0, o, jnp.exp(jnp.minimum(o, 0.0)) - 1.0)


def _attn2_body(c_ref, h1a_ref, hblk_ref, w2_ref, att2_ref, b2_ref, out_ref):
    c = c_ref[...]
    h1a = h1a_ref[...]
    w2att = lax.dot_general(w2_ref[...], att2_ref[...],
                            (((1,), (1,)), ((), ())),
                            preferred_element_type=jnp.float32)
    a_s = lax.dot_general(w2att[:, 0:1], h1a, (((0,), (1,)), ((), ())),
                          preferred_element_type=jnp.float32)
    a_d = jnp.dot(hblk_ref[...], w2att[:, 1:2],
                  preferred_element_type=jnp.float32)
    ag = jnp.max(a_s, axis=1, keepdims=True)
    t = a_d + ag
    bnd = jnp.where(t >= 0.0, t, 0.2 * t)
    u = jnp.exp(a_d - bnd)
    up = jnp.exp(0.2 * a_d - bnd)
    v = jnp.exp(a_s)
    vp = jnp.exp(0.2 * a_s)
    p = c * jnp.maximum(u * v, up * vp)
    haug = jnp.concatenate(
        [h1a, jnp.ones((N, 1), jnp.float32)], axis=1)
    a_aug = jnp.dot(p, haug, preferred_element_type=jnp.float32)
    a = a_aug[:, :D_HID] / (a_aug[:, D_HID : D_HID + 1] + 1e-16)
    z = jnp.dot(a, w2_ref[...], preferred_element_type=jnp.float32) + b2_ref[...]
    zm = z - jnp.max(z, axis=1, keepdims=True)
    out_ref[...] = zm - jnp.log(jnp.sum(jnp.exp(zm), axis=1, keepdims=True))


_SC_NC, _SC_NS = 2, 16
_ROWS = 64
_NPAD = _SC_NC * _SC_NS * _ROWS
_CHUNK = 320
_N_CHUNKS = E_EDGES // _CHUNK
_NBUF = 4


def _counts_sc_body(ed_hbm, c_hbm, b0, b1, b2, b3, s0, s1, s2, s3, acc):
    bufs = (b0, b1, b2, b3)
    sems = (s0, s1, s2, s3)
    wid = lax.axis_index("c") * _SC_NS + lax.axis_index("s")
    base = wid * _ROWS
    zeros16 = jnp.zeros((16,), jnp.float32)
    ones16 = jnp.ones((16,), jnp.float32)
    lanes = lax.iota(jnp.int32, 16)

    for b in range(_NBUF):
        pltpu.async_copy(
            ed_hbm.at[pl.ds(b * 2 * _CHUNK, 2 * _CHUNK)], bufs[b], sems[b])

    def zbody(i, carry):
        for k in range(8):
            acc[pl.ds((i * 8 + k) * 16, 16)] = zeros16
        return carry
    lax.fori_loop(0, _ROWS * N // (16 * 8), zbody, 0)

    for g in range(4):
        ln = lanes + g * 16
        col = base + ln
        plsc.addupdate_scatter(acc, [ln * N + col], ones16, mask=col < N)

    n_groups = _N_CHUNKS // _NBUF

    def group(g, carry):
        for b in range(_NBUF):
            buf = bufs[b]
            pltpu.make_async_copy(
                ed_hbm.at[pl.ds(0, 2 * _CHUNK)], buf, sems[b]).wait()

            def vbody(i, inner):
                sv = buf[pl.ds(i * 16, 16)]
                dv = buf[pl.ds(_CHUNK + i * 16, 16)]
                loc = dv - base
                m = (loc >= 0) & (loc < _ROWS)
                plsc.addupdate_scatter(acc, [loc * N + sv], ones16, mask=m)
                return inner
            lax.fori_loop(0, _CHUNK // 16, vbody, 0)

            @pl.when(g < n_groups - 1)
            def _():
                pltpu.async_copy(
                    ed_hbm.at[pl.ds((g * _NBUF + b + _NBUF) * 2 * _CHUNK,
                                    2 * _CHUNK)], buf, sems[b])
        return carry
    lax.fori_loop(0, n_groups, group, 0)

    pltpu.sync_copy(acc, c_hbm.at[pl.ds(base * N, _ROWS * N)])


def _build_counts(edge_index):
    ed = jnp.concatenate(
        [edge_index[0].reshape(_N_CHUNKS, _CHUNK),
         edge_index[1].reshape(_N_CHUNKS, _CHUNK)], axis=1).reshape(-1)
    c_flat = pl.kernel(
        _counts_sc_body,
        out_type=jax.ShapeDtypeStruct((_NPAD * N,), jnp.float32),
        mesh=plsc.VectorSubcoreMesh(
            core_axis_name="c", subcore_axis_name="s",
            num_cores=_SC_NC, num_subcores=_SC_NS),
        compiler_params=pltpu.CompilerParams(needs_layout_passes=False),
        scratch_types=(
            [pltpu.VMEM((2 * _CHUNK,), jnp.int32)] * _NBUF
            + [pltpu.SemaphoreType.DMA] * _NBUF
            + [pltpu.VMEM((_ROWS * N,), jnp.float32)]
        ),
    )(ed)
    return c_flat.reshape(_NPAD, N)


def kernel(x, edge_index, W1, att_src1, att_dst1, b1, W2, att_src2, att_dst2, b2):
    f32 = jnp.float32

    c = _build_counts(edge_index)
    att2 = jnp.concatenate([att_src2, att_dst2], axis=0)
    grid1 = (N // DB,)

    h1 = pl.pallas_call(
        _layer1_pre_body,
        grid=grid1,
        in_specs=[
            pl.BlockSpec((DB, N), lambda i: (i, 0)),
            pl.BlockSpec((N, D_HID), lambda i: (0, 0)),
        ],
        out_specs=pl.BlockSpec((DB, D_HID), lambda i: (i, 0)),
        out_shape=jax.ShapeDtypeStruct((N, D_HID), f32),
    )(x, W1)

    h1a = pl.pallas_call(
        _attn1_body,
        grid=grid1,
        in_specs=[
            pl.BlockSpec((DB, N), lambda i: (i, 0)),
            pl.BlockSpec((N, D_HID), lambda i: (0, 0)),
            pl.BlockSpec((DB, D_HID), lambda i: (i, 0)),
            pl.BlockSpec((H1, F1), lambda i: (0, 0)),
            pl.BlockSpec((H1, F1), lambda i: (0, 0)),
            pl.BlockSpec((1, D_HID), lambda i: (0, 0)),
        ],
        out_specs=pl.BlockSpec((DB, D_HID), lambda i: (i, 0)),
        out_shape=jax.ShapeDtypeStruct((N, D_HID), f32),
    )(c, h1, h1, att_src1, att_dst1, b1.reshape(1, D_HID))

    out = pl.pallas_call(
        _attn2_body,
        grid=grid1,
        in_specs=[
            pl.BlockSpec((DB, N), lambda i: (i, 0)),
            pl.BlockSpec((N, D_HID), lambda i: (0, 0)),
            pl.BlockSpec((DB, D_HID), lambda i: (i, 0)),
            pl.BlockSpec((D_HID, N), lambda i: (0, 0)),
            pl.BlockSpec((2, N), lambda i: (0, 0)),
            pl.BlockSpec((1, N), lambda i: (0, 0)),
        ],
        out_specs=pl.BlockSpec((DB, N), lambda i: (i, 0)),
        out_shape=jax.ShapeDtypeStruct((N, N), f32),
    )(c, h1a, h1a, W2, att2, b2.reshape(1, N))
    return out


# speedup vs baseline: 1.0223x; 1.0223x over previous
import functools

import jax
import jax.numpy as jnp
from jax import lax
from jax.experimental import pallas as pl
from jax.experimental.pallas import tpu as pltpu
from jax.experimental.pallas import tpu_sc as plsc

N = 2000
E_EDGES = 32000
H1, F1 = 8, 16
D_HID = H1 * F1
DB = 1000
NEG = -1e30


def _layer1_pre_body(x_ref, w1_ref, h_ref):
    h_ref[...] = jnp.dot(x_ref[...], w1_ref[...],
                         preferred_element_type=jnp.float32)


def _head_proj(att_ref):
    att = att_ref[...]
    tiled = jnp.concatenate([att] * H1, axis=1)
    row = lax.broadcasted_iota(jnp.int32, (H1, D_HID), 0)
    col = lax.broadcasted_iota(jnp.int32, (H1, D_HID), 1)
    return jnp.where(col // F1 == row, tiled, 0.0)


def _attn1_body(c_ref, h_ref, hblk_ref, asrc_ref, adst_ref, b1_ref, out_ref):
    c = c_ref[...]
    h1 = h_ref[...]
    asr = lax.dot_general(_head_proj(asrc_ref), h1, (((1,), (1,)), ((), ())),
                          preferred_element_type=jnp.float32)
    adc = lax.dot_general(hblk_ref[...], _head_proj(adst_ref),
                          (((1,), (1,)), ((), ())),
                          preferred_element_type=jnp.float32)
    haug = jnp.concatenate(
        [h1, jnp.ones((N, 1), jnp.float32)], axis=1).astype(jnp.bfloat16)
    for hh in range(H1):
        a_s = asr[hh : hh + 1, :]
        a_d = adc[:, hh : hh + 1]
        ag = jnp.max(a_s, axis=1, keepdims=True)
        t = a_d + ag
        bnd = jnp.where(t >= 0.0, t, 0.2 * t)
        u = jnp.exp(a_d - bnd)
        up = jnp.exp(0.2 * a_d - bnd)
        v = jnp.exp(a_s)
        vp = jnp.exp(0.2 * a_s)
        p = (c * jnp.maximum(u * v, up * vp)).astype(jnp.bfloat16)
        o_aug = jnp.dot(
            p, jnp.concatenate(
                [haug[:, hh * F1 : (hh + 1) * F1], haug[:, D_HID:]], axis=1),
            preferred_element_type=jnp.float32)
        o = (o_aug[:, :F1] / (o_aug[:, F1 : F1 + 1] + 1e-16)
             + b1_ref[:, hh * F1 : (hh + 1) * F1])
        out_ref[:, hh * F1 : (hh + 1) * F1] = jnp.where(
            o > 0.0, o, jnp.exp(jnp.minimum(o, 0.0)) - 1.0)


def _attn2_body(c_ref, h1a_ref, hblk_ref, w2_ref, att2_ref, b2_ref, out_ref):
    c = c_ref[...]
    h1a = h1a_ref[...]
    w2att = lax.dot_general(w2_ref[...], att2_ref[...],
                            (((1,), (1,)), ((), ())),
                            preferred_element_type=jnp.float32)
    a_s = lax.dot_general(w2att[:, 0:1], h1a, (((0,), (1,)), ((), ())),
                          preferred_element_type=jnp.float32)
    a_d = jnp.dot(hblk_ref[...], w2att[:, 1:2],
                  preferred_element_type=jnp.float32)
    ag = jnp.max(a_s, axis=1, keepdims=True)
    t = a_d + ag
    bnd = jnp.where(t >= 0.0, t, 0.2 * t)
    u = jnp.exp(a_d - bnd)
    up = jnp.exp(0.2 * a_d - bnd)
    v = jnp.exp(a_s)
    vp = jnp.exp(0.2 * a_s)
    p = (c * jnp.maximum(u * v, up * vp)).astype(jnp.bfloat16)
    haug = jnp.concatenate(
        [h1a, jnp.ones((N, 1), jnp.float32)], axis=1).astype(jnp.bfloat16)
    a_aug = jnp.dot(p, haug, preferred_element_type=jnp.float32)
    a = a_aug[:, :D_HID] / (a_aug[:, D_HID : D_HID + 1] + 1e-16)
    z = jnp.dot(a, w2_ref[...], preferred_element_type=jnp.float32) + b2_ref[...]
    zm = z - jnp.max(z, axis=1, keepdims=True)
    out_ref[...] = zm - jnp.log(jnp.sum(jnp.exp(zm), axis=1, keepdims=True))


_SC_NC, _SC_NS = 2, 16
_ROWS = 64
_NPAD = _SC_NC * _SC_NS * _ROWS
_CHUNK = 320
_N_CHUNKS = E_EDGES // _CHUNK
_NBUF = 4


def _counts_sc_body(ed_hbm, c_hbm, b0, b1, b2, b3, s0, s1, s2, s3, acc):
    bufs = (b0, b1, b2, b3)
    sems = (s0, s1, s2, s3)
    wid = lax.axis_index("c") * _SC_NS + lax.axis_index("s")
    base = wid * _ROWS
    zeros16 = jnp.zeros((16,), jnp.float32)
    ones16 = jnp.ones((16,), jnp.float32)
    lanes = lax.iota(jnp.int32, 16)

    for b in range(_NBUF):
        pltpu.async_copy(
            ed_hbm.at[pl.ds(b * 2 * _CHUNK, 2 * _CHUNK)], bufs[b], sems[b])

    def zbody(i, carry):
        for k in range(8):
            acc[pl.ds((i * 8 + k) * 16, 16)] = zeros16
        return carry
    lax.fori_loop(0, _ROWS * N // (16 * 8), zbody, 0)

    for g in range(4):
        ln = lanes + g * 16
        col = base + ln
        plsc.addupdate_scatter(acc, [ln * N + col], ones16, mask=col < N)

    n_groups = _N_CHUNKS // _NBUF

    def group(g, carry):
        for b in range(_NBUF):
            buf = bufs[b]
            pltpu.make_async_copy(
                ed_hbm.at[pl.ds(0, 2 * _CHUNK)], buf, sems[b]).wait()

            def vbody(i, inner):
                sv = buf[pl.ds(i * 16, 16)]
                dv = buf[pl.ds(_CHUNK + i * 16, 16)]
                loc = dv - base
                m = (loc >= 0) & (loc < _ROWS)
                plsc.addupdate_scatter(acc, [loc * N + sv], ones16, mask=m)
                return inner
            lax.fori_loop(0, _CHUNK // 16, vbody, 0)

            @pl.when(g < n_groups - 1)
            def _():
                pltpu.async_copy(
                    ed_hbm.at[pl.ds((g * _NBUF + b + _NBUF) * 2 * _CHUNK,
                                    2 * _CHUNK)], buf, sems[b])
        return carry
    lax.fori_loop(0, n_groups, group, 0)

    pltpu.sync_copy(acc, c_hbm.at[pl.ds(base * N, _ROWS * N)])


def _build_counts(edge_index):
    ed = jnp.concatenate(
        [edge_index[0].reshape(_N_CHUNKS, _CHUNK),
         edge_index[1].reshape(_N_CHUNKS, _CHUNK)], axis=1).reshape(-1)
    c_flat = pl.kernel(
        _counts_sc_body,
        out_type=jax.ShapeDtypeStruct((_NPAD * N,), jnp.float32),
        mesh=plsc.VectorSubcoreMesh(
            core_axis_name="c", subcore_axis_name="s",
            num_cores=_SC_NC, num_subcores=_SC_NS),
        compiler_params=pltpu.CompilerParams(needs_layout_passes=False),
        scratch_types=(
            [pltpu.VMEM((2 * _CHUNK,), jnp.int32)] * _NBUF
            + [pltpu.SemaphoreType.DMA] * _NBUF
            + [pltpu.VMEM((_ROWS * N,), jnp.float32)]
        ),
    )(ed)
    return c_flat.reshape(_NPAD, N)


def kernel(x, edge_index, W1, att_src1, att_dst1, b1, W2, att_src2, att_dst2, b2):
    f32 = jnp.float32

    c = _build_counts(edge_index)
    att2 = jnp.concatenate([att_src2, att_dst2], axis=0)
    grid1 = (N // DB,)

    h1 = pl.pallas_call(
        _layer1_pre_body,
        grid=grid1,
        in_specs=[
            pl.BlockSpec((DB, N), lambda i: (i, 0)),
            pl.BlockSpec((N, D_HID), lambda i: (0, 0)),
        ],
        out_specs=pl.BlockSpec((DB, D_HID), lambda i: (i, 0)),
        out_shape=jax.ShapeDtypeStruct((N, D_HID), f32),
    )(x, W1)

    h1a = pl.pallas_call(
        _attn1_body,
        grid=grid1,
        in_specs=[
            pl.BlockSpec((DB, N), lambda i: (i, 0)),
            pl.BlockSpec((N, D_HID), lambda i: (0, 0)),
            pl.BlockSpec((DB, D_HID), lambda i: (i, 0)),
            pl.BlockSpec((H1, F1), lambda i: (0, 0)),
            pl.BlockSpec((H1, F1), lambda i: (0, 0)),
            pl.BlockSpec((1, D_HID), lambda i: (0, 0)),
        ],
        out_specs=pl.BlockSpec((DB, D_HID), lambda i: (i, 0)),
        out_shape=jax.ShapeDtypeStruct((N, D_HID), f32),
    )(c, h1, h1, att_src1, att_dst1, b1.reshape(1, D_HID))

    out = pl.pallas_call(
        _attn2_body,
        grid=grid1,
        in_specs=[
            pl.BlockSpec((DB, N), lambda i: (i, 0)),
            pl.BlockSpec((N, D_HID), lambda i: (0, 0)),
            pl.BlockSpec((DB, D_HID), lambda i: (i, 0)),
            pl.BlockSpec((D_HID, N), lambda i: (0, 0)),
            pl.BlockSpec((2, N), lambda i: (0, 0)),
            pl.BlockSpec((1, N), lambda i: (0, 0)),
        ],
        out_specs=pl.BlockSpec((DB, N), lambda i: (i, 0)),
        out_shape=jax.ShapeDtypeStruct((N, N), f32),
    )(c, h1a, h1a, W2, att2, b2.reshape(1, N))
    return out
